# baseline (speedup 1.0000x reference)
import functools

import jax
import jax.numpy as jnp
from jax import lax
from jax.experimental import pallas as pl
from jax.experimental.pallas import tpu as pltpu
from jax.experimental.pallas import tpu_sc as plsc

B = 16384
EMB = 32
V = 1000000
TBLK = 6400


def _transpose_body(in_ref, out_ref):
    out_ref[...] = in_ref[...].T


def _transpose_table(tbl_t):
    grid = (V + TBLK - 1) // TBLK
    return pl.pallas_call(
        _transpose_body,
        grid=(grid,),
        in_specs=[pl.BlockSpec((EMB, TBLK), lambda i: (0, i))],
        out_specs=pl.BlockSpec((TBLK, EMB), lambda i: (i, 0)),
        out_shape=jax.ShapeDtypeStruct((V, EMB), jnp.float32),
    )(tbl_t)


def kernel(part_users, pos_items, neg_items, emb_users, emb_items):
    info = plsc.get_sparse_core_info()
    NC, NS = info.num_cores, info.num_subcores
    NW = NC * NS
    b_per_w = B // NW

    mesh = plsc.VectorSubcoreMesh(core_axis_name="c", subcore_axis_name="s")
    row_t = jax.ShapeDtypeStruct((B, EMB), jnp.float32)

    @functools.partial(
        pl.kernel,
        mesh=mesh,
        out_type=[row_t, row_t, row_t],
        compiler_params=pltpu.CompilerParams(use_tc_tiling_on_sc=False),
        scratch_types=[
            pltpu.VMEM((b_per_w,), jnp.int32),
            pltpu.VMEM((b_per_w,), jnp.int32),
            pltpu.VMEM((b_per_w,), jnp.int32),
            pltpu.VMEM((b_per_w, EMB), jnp.float32),
            pltpu.VMEM((b_per_w, EMB), jnp.float32),
            pltpu.VMEM((b_per_w, EMB), jnp.float32),
            pltpu.SemaphoreType.DMA,
            pltpu.SemaphoreType.DMA,
            pltpu.SemaphoreType.DMA,
        ],
    )
    def gather3(pu_hbm, pi_hbm, ni_hbm, eu_hbm, ei_hbm,
                out_u, out_p, out_n,
                idx_u, idx_p, idx_n,
                rows_u, rows_p, rows_n,
                sem_u, sem_p, sem_n):
        wid = lax.axis_index("s") * NC + lax.axis_index("c")
        base = wid * b_per_w
        pltpu.sync_copy(pu_hbm.at[pl.ds(base, b_per_w)], idx_u)
        pltpu.sync_copy(pi_hbm.at[pl.ds(base, b_per_w)], idx_p)
        pltpu.sync_copy(ni_hbm.at[pl.ds(base, b_per_w)], idx_n)
        cu = pltpu.async_copy(eu_hbm.at[idx_u], rows_u, sem_u)
        cp = pltpu.async_copy(ei_hbm.at[idx_p], rows_p, sem_p)
        cn = pltpu.async_copy(ei_hbm.at[idx_n], rows_n, sem_n)
        cu.wait()
        pltpu.sync_copy(rows_u, out_u.at[pl.ds(base, b_per_w)])
        cp.wait()
        pltpu.sync_copy(rows_p, out_p.at[pl.ds(base, b_per_w)])
        cn.wait()
        pltpu.sync_copy(rows_n, out_n.at[pl.ds(base, b_per_w)])

    eu_lin = _transpose_table(emb_users.T)
    ei_lin = _transpose_table(emb_items.T)
    out = gather3(part_users, pos_items, neg_items, eu_lin, ei_lin)
    return tuple(out)


# speedup vs baseline: 1.1034x; 1.1034x over previous
import functools

import jax
import jax.numpy as jnp
from jax import lax
from jax.experimental import pallas as pl
from jax.experimental.pallas import tpu as pltpu
from jax.experimental.pallas import tpu_sc as plsc

B = 16384
EMB = 32
V = 1000000
TBLK = 40960


def _transpose_body(in_ref, out_ref):
    out_ref[...] = in_ref[...].T


def _transpose_table(tbl_t):
    grid = (V + TBLK - 1) // TBLK
    return pl.pallas_call(
        _transpose_body,
        grid=(grid,),
        in_specs=[pl.BlockSpec((EMB, TBLK), lambda i: (0, i))],
        out_specs=pl.BlockSpec((TBLK, EMB), lambda i: (i, 0)),
        out_shape=jax.ShapeDtypeStruct((V, EMB), jnp.float32),
    )(tbl_t)


def kernel(part_users, pos_items, neg_items, emb_users, emb_items):
    info = plsc.get_sparse_core_info()
    NC, NS = info.num_cores, info.num_subcores
    NW = NC * NS
    b_per_w = B // NW

    mesh = plsc.VectorSubcoreMesh(core_axis_name="c", subcore_axis_name="s")
    row_t = jax.ShapeDtypeStruct((B, EMB), jnp.float32)

    @functools.partial(
        pl.kernel,
        mesh=mesh,
        out_type=[row_t, row_t, row_t],
        compiler_params=pltpu.CompilerParams(use_tc_tiling_on_sc=False),
        scratch_types=[
            pltpu.VMEM((b_per_w,), jnp.int32),
            pltpu.VMEM((b_per_w,), jnp.int32),
            pltpu.VMEM((b_per_w,), jnp.int32),
            pltpu.VMEM((b_per_w, EMB), jnp.float32),
            pltpu.VMEM((b_per_w, EMB), jnp.float32),
            pltpu.VMEM((b_per_w, EMB), jnp.float32),
            pltpu.SemaphoreType.DMA,
            pltpu.SemaphoreType.DMA,
            pltpu.SemaphoreType.DMA,
        ],
    )
    def gather3(pu_hbm, pi_hbm, ni_hbm, eu_hbm, ei_hbm,
                out_u, out_p, out_n,
                idx_u, idx_p, idx_n,
                rows_u, rows_p, rows_n,
                sem_u, sem_p, sem_n):
        wid = lax.axis_index("s") * NC + lax.axis_index("c")
        base = wid * b_per_w
        pltpu.sync_copy(pu_hbm.at[pl.ds(base, b_per_w)], idx_u)
        pltpu.sync_copy(pi_hbm.at[pl.ds(base, b_per_w)], idx_p)
        pltpu.sync_copy(ni_hbm.at[pl.ds(base, b_per_w)], idx_n)
        cu = pltpu.async_copy(eu_hbm.at[idx_u], rows_u, sem_u)
        cp = pltpu.async_copy(ei_hbm.at[idx_p], rows_p, sem_p)
        cn = pltpu.async_copy(ei_hbm.at[idx_n], rows_n, sem_n)
        cu.wait()
        pltpu.sync_copy(rows_u, out_u.at[pl.ds(base, b_per_w)])
        cp.wait()
        pltpu.sync_copy(rows_p, out_p.at[pl.ds(base, b_per_w)])
        cn.wait()
        pltpu.sync_copy(rows_n, out_n.at[pl.ds(base, b_per_w)])

    eu_lin = _transpose_table(emb_users.T)
    ei_lin = _transpose_table(emb_items.T)
    out = gather3(part_users, pos_items, neg_items, eu_lin, ei_lin)
    return tuple(out)


# speedup vs baseline: 1.1564x; 1.0480x over previous
import functools

import jax
import jax.numpy as jnp
from jax import lax
from jax.experimental import pallas as pl
from jax.experimental.pallas import tpu as pltpu
from jax.experimental.pallas import tpu_sc as plsc

B = 16384
EMB = 32


def kernel(part_users, pos_items, neg_items, emb_users, emb_items):
    info = plsc.get_sparse_core_info()
    NC, NS = info.num_cores, info.num_subcores
    NW = NC * NS
    b_per_w = B // NW

    mesh = plsc.VectorSubcoreMesh(core_axis_name="c", subcore_axis_name="s")
    row_t = jax.ShapeDtypeStruct((B, EMB), jnp.bfloat16)

    @functools.partial(
        pl.kernel,
        mesh=mesh,
        out_type=[row_t, row_t, row_t],
        compiler_params=pltpu.CompilerParams(use_tc_tiling_on_sc=False),
        scratch_types=[
            pltpu.VMEM((b_per_w,), jnp.int32),
            pltpu.VMEM((b_per_w,), jnp.int32),
            pltpu.VMEM((b_per_w,), jnp.int32),
            pltpu.VMEM((b_per_w, EMB), jnp.bfloat16),
            pltpu.VMEM((b_per_w, EMB), jnp.bfloat16),
            pltpu.VMEM((b_per_w, EMB), jnp.bfloat16),
            pltpu.SemaphoreType.DMA,
            pltpu.SemaphoreType.DMA,
            pltpu.SemaphoreType.DMA,
        ],
    )
    def gather3(pu_hbm, pi_hbm, ni_hbm, eu_hbm, ei_hbm,
                out_u, out_p, out_n,
                idx_u, idx_p, idx_n,
                rows_u, rows_p, rows_n,
                sem_u, sem_p, sem_n):
        wid = lax.axis_index("s") * NC + lax.axis_index("c")
        base = wid * b_per_w
        pltpu.sync_copy(pu_hbm.at[pl.ds(base, b_per_w)], idx_u)
        pltpu.sync_copy(pi_hbm.at[pl.ds(base, b_per_w)], idx_p)
        pltpu.sync_copy(ni_hbm.at[pl.ds(base, b_per_w)], idx_n)
        cu = pltpu.async_copy(eu_hbm.at[idx_u], rows_u, sem_u)
        cp = pltpu.async_copy(ei_hbm.at[idx_p], rows_p, sem_p)
        cn = pltpu.async_copy(ei_hbm.at[idx_n], rows_n, sem_n)
        cu.wait()
        pltpu.sync_copy(rows_u, out_u.at[pl.ds(base, b_per_w)])
        cp.wait()
        pltpu.sync_copy(rows_p, out_p.at[pl.ds(base, b_per_w)])
        cn.wait()
        pltpu.sync_copy(rows_n, out_n.at[pl.ds(base, b_per_w)])

    eu16 = emb_users.astype(jnp.bfloat16)
    ei16 = emb_items.astype(jnp.bfloat16)
    out = gather3(part_users, pos_items, neg_items, eu16, ei16)
    return tuple(o.astype(jnp.float32) for o in out)


# speedup vs baseline: 1.3353x; 1.1546x over previous
import functools

import jax
import jax.numpy as jnp
from jax import lax
from jax.experimental import pallas as pl
from jax.experimental.pallas import tpu as pltpu
from jax.experimental.pallas import tpu_sc as plsc

B = 16384
EMB = 32


def kernel(part_users, pos_items, neg_items, emb_users, emb_items):
    info = plsc.get_sparse_core_info()
    NC, NS = info.num_cores, info.num_subcores
    NW = NC * NS
    b_per_w = B // NW

    mesh = plsc.VectorSubcoreMesh(core_axis_name="c", subcore_axis_name="s")
    row_t = jax.ShapeDtypeStruct((B, EMB), jnp.float32)

    @functools.partial(
        pl.kernel,
        mesh=mesh,
        out_type=[row_t, row_t, row_t],
        compiler_params=pltpu.CompilerParams(use_tc_tiling_on_sc=False),
        scratch_types=[
            pltpu.VMEM((b_per_w,), jnp.int32),
            pltpu.VMEM((b_per_w,), jnp.int32),
            pltpu.VMEM((b_per_w,), jnp.int32),
            pltpu.VMEM((b_per_w, EMB), jnp.float32),
            pltpu.VMEM((b_per_w, EMB), jnp.float32),
            pltpu.VMEM((b_per_w, EMB), jnp.float32),
            pltpu.SemaphoreType.DMA,
            pltpu.SemaphoreType.DMA,
            pltpu.SemaphoreType.DMA,
        ],
    )
    def gather3(pu_hbm, pi_hbm, ni_hbm, eu_hbm, ei_hbm,
                out_u, out_p, out_n,
                idx_u, idx_p, idx_n,
                rows_u, rows_p, rows_n,
                sem_u, sem_p, sem_n):
        wid = lax.axis_index("s") * NC + lax.axis_index("c")
        base = wid * b_per_w
        pltpu.sync_copy(pu_hbm.at[pl.ds(base, b_per_w)], idx_u)
        pltpu.sync_copy(pi_hbm.at[pl.ds(base, b_per_w)], idx_p)
        pltpu.sync_copy(ni_hbm.at[pl.ds(base, b_per_w)], idx_n)
        cu = pltpu.async_copy(eu_hbm.at[idx_u], rows_u, sem_u)
        cp = pltpu.async_copy(ei_hbm.at[idx_p], rows_p, sem_p)
        cn = pltpu.async_copy(ei_hbm.at[idx_n], rows_n, sem_n)
        cu.wait()
        pltpu.sync_copy(rows_u, out_u.at[pl.ds(base, b_per_w)])
        cp.wait()
        pltpu.sync_copy(rows_p, out_p.at[pl.ds(base, b_per_w)])
        cn.wait()
        pltpu.sync_copy(rows_n, out_n.at[pl.ds(base, b_per_w)])

    out = gather3(part_users, pos_items, neg_items, emb_users, emb_items)
    return tuple(out)
